# Initial kernel scaffold; baseline (speedup 1.0000x reference)
import jax
import jax.numpy as jnp
from jax.experimental import pallas as pl


def kernel(x, means_init):
    raise NotImplementedError("write your pallas kernel here")


# speedup vs baseline: 2.2359x; 2.2359x over previous
import jax
import jax.numpy as jnp
from jax.experimental import pallas as pl
from jax.experimental.pallas import tpu as pltpu

_KMEAN_ITERS = 10
_COMMITMENT = 0.0001


def _head_kernel(x_ref, m0_ref, dists_ref, buckets_ref, loss_ref):
    n, d = x_ref.shape[1], x_ref.shape[2]
    c = m0_ref.shape[1]
    b = dists_ref.shape[0]
    l = dists_ref.shape[2]

    x = x_ref[0]
    means = m0_ref[0]

    iota_cn = jax.lax.broadcasted_iota(jnp.int32, (c, n), 0)

    def dists_of(m):
        return jax.lax.dot_general(
            x, m, (((1,), (1,)), ((), ())),
            preferred_element_type=jnp.float32)

    for _ in range(_KMEAN_ITERS):
        dists = dists_of(means)
        buckets = jnp.argmax(dists, axis=-1)
        onehot_t = (iota_cn == buckets[None, :]).astype(jnp.float32)
        sums = jax.lax.dot_general(
            onehot_t, x, (((1,), (0,)), ((), ())),
            preferred_element_type=jnp.float32,
            precision=jax.lax.Precision.HIGHEST)
        bins = jnp.sum(onehot_t, axis=1)
        norm = jnp.sqrt(jnp.sum(sums * sums, axis=-1, keepdims=True))
        means_n = sums / jnp.maximum(norm, 1e-12)
        means = jnp.where((bins == 0.0)[:, None], means, means_n)

    dists = dists_of(means)
    buckets = jnp.argmax(dists, axis=-1)

    dists_ref[:, 0, :, :] = dists.reshape(b, l, c)
    buckets_ref[0] = buckets.reshape(b, l)

    onehot_t = (iota_cn == buckets[None, :]).astype(jnp.float32)
    routed = jax.lax.dot_general(
        onehot_t, means, (((0,), (0,)), ((), ())),
        preferred_element_type=jnp.float32,
        precision=jax.lax.Precision.HIGHEST)
    resid = x - routed
    loss_ref[0, 0, 0] = jnp.sum(resid * resid)


def kernel(x, means_init):
    b, h, l, d = x.shape
    c = means_init.shape[1]
    n = b * l

    flat = jnp.swapaxes(x, 0, 1).reshape(h, n, d)
    idx = jax.random.permutation(jax.random.key(42), n)[:c]
    means0 = flat[:, idx]

    dists, buckets_hbl, loss_parts = pl.pallas_call(
        _head_kernel,
        grid=(h,),
        in_specs=[
            pl.BlockSpec((1, n, d), lambda i: (i, 0, 0)),
            pl.BlockSpec((1, c, d), lambda i: (i, 0, 0)),
        ],
        out_specs=[
            pl.BlockSpec((b, 1, l, c), lambda i: (0, i, 0, 0)),
            pl.BlockSpec((1, b, l), lambda i: (i, 0, 0)),
            pl.BlockSpec((1, 1, 1), lambda i: (i, 0, 0), memory_space=pltpu.SMEM),
        ],
        out_shape=[
            jax.ShapeDtypeStruct((b, h, l, c), jnp.float32),
            jax.ShapeDtypeStruct((h, b, l), jnp.int32),
            jax.ShapeDtypeStruct((h, 1, 1), jnp.float32),
        ],
        compiler_params=pltpu.CompilerParams(
            dimension_semantics=("parallel",),
        ),
    )(flat, means0)

    buckets = jnp.swapaxes(buckets_hbl, 0, 1)
    loss = jnp.sum(loss_parts) / (b * h * l * d) * _COMMITMENT
    return dists, buckets, loss


# speedup vs baseline: 5.3135x; 2.3765x over previous
import jax
import jax.numpy as jnp
from jax.experimental import pallas as pl
from jax.experimental.pallas import tpu as pltpu

_KMEAN_ITERS = 10
_COMMITMENT = 0.0001


def _head_kernel(x_ref, m0_ref, dists_ref, buckets_ref, loss_ref):
    b = x_ref.shape[0]
    l, d = x_ref.shape[2], x_ref.shape[3]
    c = m0_ref.shape[1]
    n = b * l

    x = x_ref[:, 0, :, :].reshape(n, d)
    means = m0_ref[0]

    x_hi = x.astype(jnp.bfloat16)
    r = x - x_hi.astype(jnp.float32)
    x_mid = r.astype(jnp.bfloat16)
    x_lo = (r - x_mid.astype(jnp.float32)).astype(jnp.bfloat16)
    ones = jnp.ones((n, 1), dtype=jnp.bfloat16)
    rhs = jnp.concatenate([x_hi, x_mid, x_lo, ones], axis=1)

    iota_cn = jax.lax.broadcasted_iota(jnp.int32, (c, n), 0)

    def dists_of(m):
        return jax.lax.dot_general(
            x, m, (((1,), (1,)), ((), ())),
            preferred_element_type=jnp.float32)

    for _ in range(_KMEAN_ITERS):
        dists = dists_of(means)
        buckets = jnp.argmax(dists, axis=-1)
        onehot_t = jnp.where(
            iota_cn == buckets[None, :], 1.0, 0.0).astype(jnp.bfloat16)
        parts = jax.lax.dot_general(
            onehot_t, rhs, (((1,), (0,)), ((), ())),
            preferred_element_type=jnp.float32)
        sums = (parts[:, :d] + parts[:, d:2 * d]) + parts[:, 2 * d:3 * d]
        bins = parts[:, 3 * d]
        norm = jnp.sqrt(jnp.sum(sums * sums, axis=-1, keepdims=True))
        means_n = sums / jnp.maximum(norm, 1e-12)
        means = jnp.where((bins == 0.0)[:, None], means, means_n)

    dists = dists_of(means)
    buckets = jnp.argmax(dists, axis=-1)

    dists_ref[:, 0, :, :] = dists.reshape(b, l, c)
    buckets_ref[0] = buckets.reshape(b, l)

    onehot_t = jnp.where(iota_cn == buckets[None, :], 1.0, 0.0)
    routed = jax.lax.dot_general(
        onehot_t, means, (((0,), (0,)), ((), ())),
        preferred_element_type=jnp.float32,
        precision=jax.lax.Precision.HIGHEST)
    resid = x - routed
    loss_ref[0, 0, 0] = jnp.sum(resid * resid)


def kernel(x, means_init):
    b, h, l, d = x.shape
    c = means_init.shape[1]
    n = b * l

    idx = jax.random.permutation(jax.random.key(42), n)[:c]
    means0 = jnp.transpose(x[idx // l, :, idx % l, :], (1, 0, 2))

    dists, buckets_hbl, loss_parts = pl.pallas_call(
        _head_kernel,
        grid=(h,),
        in_specs=[
            pl.BlockSpec((b, 1, l, d), lambda i: (0, i, 0, 0)),
            pl.BlockSpec((1, c, d), lambda i: (i, 0, 0)),
        ],
        out_specs=[
            pl.BlockSpec((b, 1, l, c), lambda i: (0, i, 0, 0)),
            pl.BlockSpec((1, b, l), lambda i: (i, 0, 0)),
            pl.BlockSpec((1, 1, 1), lambda i: (i, 0, 0), memory_space=pltpu.SMEM),
        ],
        out_shape=[
            jax.ShapeDtypeStruct((b, h, l, c), jnp.float32),
            jax.ShapeDtypeStruct((h, b, l), jnp.int32),
            jax.ShapeDtypeStruct((h, 1, 1), jnp.float32),
        ],
        compiler_params=pltpu.CompilerParams(
            dimension_semantics=("parallel",),
        ),
    )(x, means0)

    buckets = jnp.swapaxes(buckets_hbl, 0, 1)
    loss = jnp.sum(loss_parts) / (b * h * l * d) * _COMMITMENT
    return dists, buckets, loss


# speedup vs baseline: 8.5788x; 1.6145x over previous
import jax
import jax.numpy as jnp
from jax.experimental import pallas as pl
from jax.experimental.pallas import tpu as pltpu

_KMEAN_ITERS = 10
_COMMITMENT = 0.0001


def _head_kernel(x_ref, m0_ref, dists_ref, buckets_ref, loss_ref):
    b = x_ref.shape[0]
    l, d = x_ref.shape[2], x_ref.shape[3]
    c = m0_ref.shape[1]
    n = b * l

    x = x_ref[:, 0, :, :].reshape(n, d)
    means = m0_ref[0]

    x_hi = x.astype(jnp.bfloat16)
    r = x - x_hi.astype(jnp.float32)
    x_mid = r.astype(jnp.bfloat16)
    x_lo = (r - x_mid.astype(jnp.float32)).astype(jnp.bfloat16)
    ones = jnp.ones((n, 1), dtype=jnp.bfloat16)
    rhs = jnp.concatenate([x_hi, x_mid, x_lo, ones], axis=1)

    iota_cn = jax.lax.broadcasted_iota(jnp.int32, (c, n), 0)

    def onehot_and_parts(buckets):
        onehot_t = jnp.where(
            iota_cn == buckets[None, :], 1.0, 0.0).astype(jnp.bfloat16)
        parts = jax.lax.dot_general(
            onehot_t, rhs, (((1,), (0,)), ((), ())),
            preferred_element_type=jnp.float32)
        sums = (parts[:, :d] + parts[:, d:2 * d]) + parts[:, 2 * d:3 * d]
        bins = parts[:, 3 * d]
        return sums, bins

    for _ in range(_KMEAN_ITERS):
        dists_t = jax.lax.dot_general(
            means, x, (((1,), (1,)), ((), ())),
            preferred_element_type=jnp.float32)
        buckets = jnp.argmax(dists_t, axis=0)
        sums, bins = onehot_and_parts(buckets)
        norm = jnp.sqrt(jnp.sum(sums * sums, axis=-1, keepdims=True))
        means_n = sums / jnp.maximum(norm, 1e-12)
        means = jnp.where((bins == 0.0)[:, None], means, means_n)

    dists = jax.lax.dot_general(
        x, means, (((1,), (1,)), ((), ())),
        preferred_element_type=jnp.float32)
    buckets = jnp.argmax(dists, axis=-1)

    dists_ref[:, 0, :, :] = dists.reshape(b, l, c)
    buckets_ref[0] = buckets.reshape(b, l)

    sums_f, bins_f = onehot_and_parts(buckets)
    xsq = jnp.sum(x * x)
    cross = jnp.sum(sums_f * means)
    msq = jnp.sum(means * means, axis=-1)
    loss_ref[0, 0, 0] = xsq - 2.0 * cross + jnp.sum(bins_f * msq)


def kernel(x, means_init):
    b, h, l, d = x.shape
    c = means_init.shape[1]
    n = b * l

    idx = jax.random.permutation(jax.random.key(42), n)[:c]
    means0 = jnp.transpose(x[idx // l, :, idx % l, :], (1, 0, 2))

    dists, buckets_hbl, loss_parts = pl.pallas_call(
        _head_kernel,
        grid=(h,),
        in_specs=[
            pl.BlockSpec((b, 1, l, d), lambda i: (0, i, 0, 0)),
            pl.BlockSpec((1, c, d), lambda i: (i, 0, 0)),
        ],
        out_specs=[
            pl.BlockSpec((b, 1, l, c), lambda i: (0, i, 0, 0)),
            pl.BlockSpec((1, b, l), lambda i: (i, 0, 0)),
            pl.BlockSpec((1, 1, 1), lambda i: (i, 0, 0), memory_space=pltpu.SMEM),
        ],
        out_shape=[
            jax.ShapeDtypeStruct((b, h, l, c), jnp.float32),
            jax.ShapeDtypeStruct((h, b, l), jnp.int32),
            jax.ShapeDtypeStruct((h, 1, 1), jnp.float32),
        ],
        compiler_params=pltpu.CompilerParams(
            dimension_semantics=("parallel",),
        ),
    )(x, means0)

    buckets = jnp.swapaxes(buckets_hbl, 0, 1)
    loss = jnp.sum(loss_parts) / (b * h * l * d) * _COMMITMENT
    return dists, buckets, loss


# speedup vs baseline: 9.0879x; 1.0594x over previous
import jax
import jax.numpy as jnp
import numpy as np
from jax.experimental import pallas as pl
from jax.experimental.pallas import tpu as pltpu

_KMEAN_ITERS = 10
_COMMITMENT = 0.0001


def _head_kernel(x_ref, m0_ref, dists_ref, buckets_ref, loss_ref):
    b = x_ref.shape[0]
    l, d = x_ref.shape[2], x_ref.shape[3]
    c = m0_ref.shape[1]
    n = b * l

    x = x_ref[:, 0, :, :].reshape(n, d)
    means = m0_ref[0]

    x_hi = x.astype(jnp.bfloat16)
    r = x - x_hi.astype(jnp.float32)
    x_mid = r.astype(jnp.bfloat16)
    x_lo = (r - x_mid.astype(jnp.float32)).astype(jnp.bfloat16)
    ones = jnp.ones((n, 1), dtype=jnp.bfloat16)
    rhs = jnp.concatenate([x_hi, x_mid, x_lo, ones], axis=1)

    iota_cn = jax.lax.broadcasted_iota(jnp.int32, (c, n), 0)

    def onehot_and_parts(buckets):
        onehot_t = jnp.where(
            iota_cn == buckets[None, :], 1.0, 0.0).astype(jnp.bfloat16)
        parts = jax.lax.dot_general(
            onehot_t, rhs, (((1,), (0,)), ((), ())),
            preferred_element_type=jnp.float32)
        sums = (parts[:, :d] + parts[:, d:2 * d]) + parts[:, 2 * d:3 * d]
        bins = parts[:, 3 * d]
        return sums, bins

    for _ in range(_KMEAN_ITERS):
        dists_t = jax.lax.dot_general(
            means, x, (((1,), (1,)), ((), ())),
            preferred_element_type=jnp.float32)
        buckets = jnp.argmax(dists_t, axis=0)
        sums, bins = onehot_and_parts(buckets)
        norm = jnp.sqrt(jnp.sum(sums * sums, axis=-1, keepdims=True))
        means_n = sums / jnp.maximum(norm, 1e-12)
        means = jnp.where((bins == 0.0)[:, None], means, means_n)

    dists = jax.lax.dot_general(
        x, means, (((1,), (1,)), ((), ())),
        preferred_element_type=jnp.float32)
    buckets = jnp.argmax(dists, axis=-1)

    dists_ref[:, 0, :, :] = dists.reshape(b, l, c)
    buckets_ref[0] = buckets.reshape(b, l)

    sums_f, bins_f = onehot_and_parts(buckets)
    xsq = jnp.sum(x * x)
    cross = jnp.sum(sums_f * means)
    msq = jnp.sum(means * means, axis=-1)
    loss_ref[0, 0, 0] = xsq - 2.0 * cross + jnp.sum(bins_f * msq)


def kernel(x, means_init):
    b, h, l, d = x.shape
    c = means_init.shape[1]
    n = b * l

    with jax.ensure_compile_time_eval():
        idx = np.asarray(jax.random.permutation(jax.random.key(42), n)[:c])
    means0 = jnp.transpose(x[idx // l, :, idx % l, :], (1, 0, 2))

    dists, buckets_hbl, loss_parts = pl.pallas_call(
        _head_kernel,
        grid=(h,),
        in_specs=[
            pl.BlockSpec((b, 1, l, d), lambda i: (0, i, 0, 0)),
            pl.BlockSpec((1, c, d), lambda i: (i, 0, 0)),
        ],
        out_specs=[
            pl.BlockSpec((b, 1, l, c), lambda i: (0, i, 0, 0)),
            pl.BlockSpec((1, b, l), lambda i: (i, 0, 0)),
            pl.BlockSpec((1, 1, 1), lambda i: (i, 0, 0), memory_space=pltpu.SMEM),
        ],
        out_shape=[
            jax.ShapeDtypeStruct((b, h, l, c), jnp.float32),
            jax.ShapeDtypeStruct((h, b, l), jnp.int32),
            jax.ShapeDtypeStruct((h, 1, 1), jnp.float32),
        ],
        compiler_params=pltpu.CompilerParams(
            dimension_semantics=("parallel",),
        ),
    )(x, means0)

    buckets = jnp.swapaxes(buckets_hbl, 0, 1)
    loss = jnp.sum(loss_parts) / (b * h * l * d) * _COMMITMENT
    return dists, buckets, loss


# speedup vs baseline: 9.1949x; 1.0118x over previous
import jax
import jax.numpy as jnp
import numpy as np
from jax.experimental import pallas as pl
from jax.experimental.pallas import tpu as pltpu

_KMEAN_ITERS = 10
_COMMITMENT = 0.0001
_HEADS_PER_BLOCK = 2


def _heads_kernel(x_ref, m0_ref, dists_ref, buckets_ref, loss_ref):
    b = x_ref.shape[0]
    l, d = x_ref.shape[2], x_ref.shape[3]
    c = m0_ref.shape[1]
    n = b * l

    iota_cn = jax.lax.broadcasted_iota(jnp.int32, (c, n), 0)

    for j in range(_HEADS_PER_BLOCK):
        x = x_ref[:, j, :, :].reshape(n, d)
        means = m0_ref[j]

        x_hi = x.astype(jnp.bfloat16)
        r = x - x_hi.astype(jnp.float32)
        x_mid = r.astype(jnp.bfloat16)
        x_lo = (r - x_mid.astype(jnp.float32)).astype(jnp.bfloat16)
        ones = jnp.ones((n, 1), dtype=jnp.bfloat16)
        rhs = jnp.concatenate([x_hi, x_mid, x_lo, ones], axis=1)

        def onehot_and_parts(buckets):
            onehot_t = jnp.where(
                iota_cn == buckets[None, :], 1.0, 0.0).astype(jnp.bfloat16)
            parts = jax.lax.dot_general(
                onehot_t, rhs, (((1,), (0,)), ((), ())),
                preferred_element_type=jnp.float32)
            sums = (parts[:, :d] + parts[:, d:2 * d]) + parts[:, 2 * d:3 * d]
            bins = parts[:, 3 * d]
            return sums, bins

        for _ in range(_KMEAN_ITERS):
            dists_t = jax.lax.dot_general(
                means, x, (((1,), (1,)), ((), ())),
                preferred_element_type=jnp.float32)
            buckets = jnp.argmax(dists_t, axis=0)
            sums, bins = onehot_and_parts(buckets)
            norm = jnp.sqrt(jnp.sum(sums * sums, axis=-1, keepdims=True))
            means_n = sums / jnp.maximum(norm, 1e-12)
            means = jnp.where((bins == 0.0)[:, None], means, means_n)

        dists = jax.lax.dot_general(
            x, means, (((1,), (1,)), ((), ())),
            preferred_element_type=jnp.float32)
        buckets = jnp.argmax(dists, axis=-1)

        dists_ref[:, j, :, :] = dists.reshape(b, l, c)
        buckets_ref[j] = buckets.reshape(b, l)

        sums_f, bins_f = onehot_and_parts(buckets)
        xsq = jnp.sum(x * x)
        cross = jnp.sum(sums_f * means)
        msq = jnp.sum(means * means, axis=-1)
        loss_ref[j, 0, 0] = xsq - 2.0 * cross + jnp.sum(bins_f * msq)


def kernel(x, means_init):
    b, h, l, d = x.shape
    c = means_init.shape[1]
    n = b * l
    hpb = _HEADS_PER_BLOCK

    with jax.ensure_compile_time_eval():
        idx = np.asarray(jax.random.permutation(jax.random.key(42), n)[:c])
    means0 = jnp.transpose(x[idx // l, :, idx % l, :], (1, 0, 2))

    dists, buckets_hbl, loss_parts = pl.pallas_call(
        _heads_kernel,
        grid=(h // hpb,),
        in_specs=[
            pl.BlockSpec((b, hpb, l, d), lambda i: (0, i, 0, 0)),
            pl.BlockSpec((hpb, c, d), lambda i: (i, 0, 0)),
        ],
        out_specs=[
            pl.BlockSpec((b, hpb, l, c), lambda i: (0, i, 0, 0)),
            pl.BlockSpec((hpb, b, l), lambda i: (i, 0, 0)),
            pl.BlockSpec((hpb, 1, 1), lambda i: (i, 0, 0), memory_space=pltpu.SMEM),
        ],
        out_shape=[
            jax.ShapeDtypeStruct((b, h, l, c), jnp.float32),
            jax.ShapeDtypeStruct((h, b, l), jnp.int32),
            jax.ShapeDtypeStruct((h, 1, 1), jnp.float32),
        ],
        compiler_params=pltpu.CompilerParams(
            dimension_semantics=("parallel",),
        ),
    )(x, means0)

    buckets = jnp.swapaxes(buckets_hbl, 0, 1)
    loss = jnp.sum(loss_parts) / (b * h * l * d) * _COMMITMENT
    return dists, buckets, loss


# speedup vs baseline: 10.4112x; 1.1323x over previous
import jax
import jax.numpy as jnp
import numpy as np
from jax.experimental import pallas as pl
from jax.experimental.pallas import tpu as pltpu

_KMEAN_ITERS = 10
_COMMITMENT = 0.0001
_HEADS_PER_BLOCK = 2


def _heads_kernel(x_ref, m0_ref, dists_ref, buckets_ref, loss_ref):
    b = x_ref.shape[0]
    l, d = x_ref.shape[2], x_ref.shape[3]
    c = m0_ref.shape[1]
    n = b * l

    iota_cn = jax.lax.broadcasted_iota(jnp.int32, (c, n), 0)
    hpb = _HEADS_PER_BLOCK

    xs, rhss, means = [], [], []
    for j in range(hpb):
        x = x_ref[:, j, :, :].reshape(n, d)
        x_hi = x.astype(jnp.bfloat16)
        r = x - x_hi.astype(jnp.float32)
        x_mid = r.astype(jnp.bfloat16)
        x_lo = (r - x_mid.astype(jnp.float32)).astype(jnp.bfloat16)
        ones = jnp.ones((n, 1), dtype=jnp.bfloat16)
        xs.append(x)
        rhss.append(jnp.concatenate([x_hi, x_mid, x_lo, ones], axis=1))
        means.append(m0_ref[j])

    def onehot_and_parts(buckets, rhs):
        onehot_t = jnp.where(
            iota_cn == buckets[None, :], 1.0, 0.0).astype(jnp.bfloat16)
        parts = jax.lax.dot_general(
            onehot_t, rhs, (((1,), (0,)), ((), ())),
            preferred_element_type=jnp.float32)
        sums = (parts[:, :d] + parts[:, d:2 * d]) + parts[:, 2 * d:3 * d]
        bins = parts[:, 3 * d]
        return sums, bins

    for _ in range(_KMEAN_ITERS):
        dists_t = [jax.lax.dot_general(
            means[j], xs[j], (((1,), (1,)), ((), ())),
            preferred_element_type=jnp.float32) for j in range(hpb)]
        buckets = [jnp.argmax(dists_t[j], axis=0) for j in range(hpb)]
        sums_bins = [onehot_and_parts(buckets[j], rhss[j]) for j in range(hpb)]
        for j in range(hpb):
            sums, bins = sums_bins[j]
            norm = jnp.sqrt(jnp.sum(sums * sums, axis=-1, keepdims=True))
            means_n = sums / jnp.maximum(norm, 1e-12)
            means[j] = jnp.where((bins == 0.0)[:, None], means[j], means_n)

    dists = [jax.lax.dot_general(
        xs[j], means[j], (((1,), (1,)), ((), ())),
        preferred_element_type=jnp.float32) for j in range(hpb)]
    buckets = [jnp.argmax(dists[j], axis=-1) for j in range(hpb)]

    for j in range(hpb):
        dists_ref[:, j, :, :] = dists[j].reshape(b, l, c)
        buckets_ref[j] = buckets[j].reshape(b, l)

        sums_f, bins_f = onehot_and_parts(buckets[j], rhss[j])
        xsq = jnp.sum(xs[j] * xs[j])
        cross = jnp.sum(sums_f * means[j])
        msq = jnp.sum(means[j] * means[j], axis=-1)
        loss_ref[j, 0, 0] = xsq - 2.0 * cross + jnp.sum(bins_f * msq)


def kernel(x, means_init):
    b, h, l, d = x.shape
    c = means_init.shape[1]
    n = b * l
    hpb = _HEADS_PER_BLOCK

    with jax.ensure_compile_time_eval():
        idx = np.asarray(jax.random.permutation(jax.random.key(42), n)[:c])
    means0 = jnp.transpose(x[idx // l, :, idx % l, :], (1, 0, 2))

    dists, buckets_hbl, loss_parts = pl.pallas_call(
        _heads_kernel,
        grid=(h // hpb,),
        in_specs=[
            pl.BlockSpec((b, hpb, l, d), lambda i: (0, i, 0, 0)),
            pl.BlockSpec((hpb, c, d), lambda i: (i, 0, 0)),
        ],
        out_specs=[
            pl.BlockSpec((b, hpb, l, c), lambda i: (0, i, 0, 0)),
            pl.BlockSpec((hpb, b, l), lambda i: (i, 0, 0)),
            pl.BlockSpec((hpb, 1, 1), lambda i: (i, 0, 0), memory_space=pltpu.SMEM),
        ],
        out_shape=[
            jax.ShapeDtypeStruct((b, h, l, c), jnp.float32),
            jax.ShapeDtypeStruct((h, b, l), jnp.int32),
            jax.ShapeDtypeStruct((h, 1, 1), jnp.float32),
        ],
        compiler_params=pltpu.CompilerParams(
            dimension_semantics=("parallel",),
        ),
    )(x, means0)

    buckets = jnp.swapaxes(buckets_hbl, 0, 1)
    loss = jnp.sum(loss_parts) / (b * h * l * d) * _COMMITMENT
    return dists, buckets, loss


# speedup vs baseline: 11.0074x; 1.0573x over previous
import jax
import jax.numpy as jnp
import numpy as np
from jax.experimental import pallas as pl
from jax.experimental.pallas import tpu as pltpu

_KMEAN_ITERS = 10
_COMMITMENT = 0.0001
_HEADS_PER_BLOCK = 2


def _make_heads_kernel(with_idx):
    def body(x_ref, m0_ref, idx_ref, dists_ref, buckets_ref, loss_ref):
        b = x_ref.shape[0]
        l, d = x_ref.shape[2], x_ref.shape[3]
        n = b * l
        c = idx_ref.shape[0] if with_idx else m0_ref.shape[1]
        hpb = _HEADS_PER_BLOCK

        iota_cn = jax.lax.broadcasted_iota(jnp.int32, (c, n), 0)

        def exact_parts_matmul(lhs_bf16, rhs):
            parts = jax.lax.dot_general(
                lhs_bf16, rhs, (((1,), (0,)), ((), ())),
                preferred_element_type=jnp.float32)
            sums = (parts[:, :d] + parts[:, d:2 * d]) + parts[:, 2 * d:3 * d]
            return sums, parts[:, 3 * d]

        if with_idx:
            iota_n = jax.lax.broadcasted_iota(jnp.int32, (c, n), 1)
            sel = jnp.where(idx_ref[:, :] == iota_n,
                            1.0, 0.0).astype(jnp.bfloat16)

        xs, rhss, means = [], [], []
        for j in range(hpb):
            x = x_ref[:, j, :, :].reshape(n, d)
            x_hi = x.astype(jnp.bfloat16)
            r = x - x_hi.astype(jnp.float32)
            x_mid = r.astype(jnp.bfloat16)
            x_lo = (r - x_mid.astype(jnp.float32)).astype(jnp.bfloat16)
            ones = jnp.ones((n, 1), dtype=jnp.bfloat16)
            rhs = jnp.concatenate([x_hi, x_mid, x_lo, ones], axis=1)
            if with_idx:
                m0, _ = exact_parts_matmul(sel, rhs)
            else:
                m0 = m0_ref[j]
            xs.append(x)
            rhss.append(rhs)
            means.append(m0)

        def onehot_and_parts(buckets, rhs):
            onehot_t = jnp.where(
                iota_cn == buckets[None, :], 1.0, 0.0).astype(jnp.bfloat16)
            return exact_parts_matmul(onehot_t, rhs)

        for _ in range(_KMEAN_ITERS):
            dists_t = [jax.lax.dot_general(
                means[j], xs[j], (((1,), (1,)), ((), ())),
                preferred_element_type=jnp.float32) for j in range(hpb)]
            buckets = [jnp.argmax(dists_t[j], axis=0) for j in range(hpb)]
            sums_bins = [onehot_and_parts(buckets[j], rhss[j])
                         for j in range(hpb)]
            for j in range(hpb):
                sums, bins = sums_bins[j]
                norm = jnp.sqrt(jnp.sum(sums * sums, axis=-1, keepdims=True))
                means_n = sums / jnp.maximum(norm, 1e-12)
                means[j] = jnp.where((bins == 0.0)[:, None], means[j], means_n)

        dists = [jax.lax.dot_general(
            xs[j], means[j], (((1,), (1,)), ((), ())),
            preferred_element_type=jnp.float32) for j in range(hpb)]
        buckets = [jnp.argmax(dists[j], axis=-1) for j in range(hpb)]

        for j in range(hpb):
            dists_ref[:, j, :, :] = dists[j].reshape(b, l, c)
            buckets_ref[:, 0, j, :] = buckets[j].reshape(b, l)

            sums_f, bins_f = onehot_and_parts(buckets[j], rhss[j])
            xsq = jnp.sum(xs[j] * xs[j])
            cross = jnp.sum(sums_f * means[j])
            msq = jnp.sum(means[j] * means[j], axis=-1)
            loss_ref[j, 0, 0] = xsq - 2.0 * cross + jnp.sum(bins_f * msq)

    if with_idx:
        def kernel_with_idx(x_ref, idx_ref, dists_ref, buckets_ref, loss_ref):
            body(x_ref, None, idx_ref, dists_ref, buckets_ref, loss_ref)
        return kernel_with_idx

    def kernel_with_m0(x_ref, m0_ref, dists_ref, buckets_ref, loss_ref):
        body(x_ref, m0_ref, None, dists_ref, buckets_ref, loss_ref)
    return kernel_with_m0


def kernel(x, means_init):
    b, h, l, d = x.shape
    c = means_init.shape[1]
    n = b * l
    hpb = _HEADS_PER_BLOCK

    try:
        with jax.ensure_compile_time_eval():
            idx = np.asarray(jax.random.permutation(jax.random.key(42), n)[:c])
    except Exception:
        idx = None

    if idx is not None:
        kernel_fn = _make_heads_kernel(True)
        operands = (x, jnp.asarray(idx.reshape(c, 1), dtype=jnp.int32))
        in_specs = [
            pl.BlockSpec((b, hpb, l, d), lambda i: (0, i, 0, 0)),
            pl.BlockSpec((c, 1), lambda i: (0, 0)),
        ]
    else:
        idxt = jax.random.permutation(jax.random.key(42), n)[:c]
        means0 = jnp.transpose(x[idxt // l, :, idxt % l, :], (1, 0, 2))
        kernel_fn = _make_heads_kernel(False)
        operands = (x, means0)
        in_specs = [
            pl.BlockSpec((b, hpb, l, d), lambda i: (0, i, 0, 0)),
            pl.BlockSpec((hpb, c, d), lambda i: (i, 0, 0)),
        ]

    dists, buckets_r, loss_parts = pl.pallas_call(
        kernel_fn,
        grid=(h // hpb,),
        in_specs=in_specs,
        out_specs=[
            pl.BlockSpec((b, hpb, l, c), lambda i: (0, i, 0, 0)),
            pl.BlockSpec((b, 1, hpb, l), lambda i: (0, i, 0, 0)),
            pl.BlockSpec((hpb, 1, 1), lambda i: (i, 0, 0),
                         memory_space=pltpu.SMEM),
        ],
        out_shape=[
            jax.ShapeDtypeStruct((b, h, l, c), jnp.float32),
            jax.ShapeDtypeStruct((b, h // hpb, hpb, l), jnp.int32),
            jax.ShapeDtypeStruct((h, 1, 1), jnp.float32),
        ],
        compiler_params=pltpu.CompilerParams(
            dimension_semantics=("parallel",),
        ),
    )(*operands)

    buckets = buckets_r.reshape(b, h, l)
    loss = jnp.sum(loss_parts) / (b * h * l * d) * _COMMITMENT
    return dists, buckets, loss
